# Initial kernel scaffold; baseline (speedup 1.0000x reference)
import jax
import jax.numpy as jnp
from jax.experimental import pallas as pl


def kernel(x, edge_index, batch, W0, b0, Wc, bc, gamma, beta, W1, b1, W2, b2, W3, b3):
    raise NotImplementedError("write your pallas kernel here")


# speedup vs baseline: 7.0234x; 7.0234x over previous
import functools

import jax
import jax.numpy as jnp
from jax import lax
from jax.experimental import pallas as pl
from jax.experimental.pallas import tpu as pltpu
from jax.experimental.pallas import tpu_sc as plsc

F32 = jnp.float32
NUM_GRAPHS = 64


def _elu(v):
    return jnp.where(v > 0, v, jnp.exp(jnp.minimum(v, 0.0)) - 1.0)


def _in_proj_body(x_ref, w_ref, b_ref, o_ref):
    o_ref[...] = _elu(
        jnp.dot(x_ref[...], w_ref[...], preferred_element_type=F32) + b_ref[...]
    )


def _mm_split_body(h_ref, w_ref, d0_ref, d1_ref, o0_ref, o1_ref):
    dis = lax.rsqrt(d0_ref[...] + d1_ref[...] + 1.0)
    xw = jnp.dot(h_ref[...], w_ref[...], preferred_element_type=F32) * dis
    hh = xw.shape[1] // 2
    o0_ref[...] = xw[:, :hh]
    o1_ref[...] = xw[:, hh:]


def _post(s0, s1, xp0, xp1, d0, d1, bc, gm, bt):
    dis = lax.rsqrt(d0 + d1 + 1.0)
    p = jnp.concatenate([s0 + xp0, s1 + xp1], axis=1)
    agg = p * dis + bc
    e = _elu(agg)
    mu = jnp.mean(e, axis=1, keepdims=True)
    var = jnp.mean((e - mu) ** 2, axis=1, keepdims=True)
    return (e - mu) * lax.rsqrt(var + 1e-5) * gm + bt


def _step_body(s0_ref, s1_ref, xp0_ref, xp1_ref, d0_ref, d1_ref, bc_ref,
               gm_ref, bt_ref, w_ref, o0_ref, o1_ref):
    h = _post(s0_ref[...], s1_ref[...], xp0_ref[...], xp1_ref[...],
              d0_ref[...], d1_ref[...], bc_ref[...], gm_ref[...], bt_ref[...])
    dis = lax.rsqrt(d0_ref[...] + d1_ref[...] + 1.0)
    xw = jnp.dot(h, w_ref[...], preferred_element_type=F32) * dis
    hh = xw.shape[1] // 2
    o0_ref[...] = xw[:, :hh]
    o1_ref[...] = xw[:, hh:]


def _post_body(s0_ref, s1_ref, xp0_ref, xp1_ref, d0_ref, d1_ref, bc_ref,
               gm_ref, bt_ref, o_ref):
    o_ref[...] = _post(s0_ref[...], s1_ref[...], xp0_ref[...], xp1_ref[...],
                       d0_ref[...], d1_ref[...], bc_ref[...], gm_ref[...],
                       bt_ref[...])


def _pool_body(h_ref, batch_ref, sums_ref, maxs_ref, cnts_ref):
    step = pl.program_id(0)
    b = batch_ref[...]
    hdim = h_ref.shape[1]
    sm_list, mx_list, ct_list = [], [], []
    for j in range(8):
        g = step * 8 + j
        s = jnp.sum((b < g).astype(jnp.int32))
        e = jnp.sum((b < g + 1).astype(jnp.int32))
        c0 = s // 8
        c1 = (e + 7) // 8

        def body(i, carry, s=s, e=e):
            sm, mx = carry
            blk = h_ref[pl.ds(i * 8, 8), :]
            rid = lax.broadcasted_iota(jnp.int32, (8, 1), 0) + i * 8
            m = (rid >= s) & (rid < e)
            sm = sm + jnp.sum(jnp.where(m, blk, 0.0), axis=0, keepdims=True)
            mx = jnp.maximum(
                mx, jnp.max(jnp.where(m, blk, -jnp.inf), axis=0, keepdims=True))
            return sm, mx

        sm, mx = lax.fori_loop(
            c0, c1, body,
            (jnp.zeros((1, hdim), F32), jnp.full((1, hdim), -jnp.inf, F32)))
        sm_list.append(sm)
        mx_list.append(mx)
        ct_list.append((e - s).astype(F32).reshape(1, 1))
    sums_ref[...] = jnp.concatenate(sm_list, axis=0)
    maxs_ref[...] = jnp.concatenate(mx_list, axis=0)
    cnts_ref[...] = jnp.concatenate(ct_list, axis=0)


def _mlp_body(sm_ref, mx_ref, ct_ref, w1_ref, b1_ref, w2_ref, b2_ref,
              w3_ref, b3_ref, o_ref):
    mean = sm_ref[...] / jnp.maximum(ct_ref[...], 1.0)
    hp = jnp.concatenate([mean, mx_ref[...]], axis=1)
    z = _elu(jnp.dot(hp, w1_ref[...], preferred_element_type=F32) + b1_ref[...])
    z = _elu(jnp.dot(z, w2_ref[...], preferred_element_type=F32) + b2_ref[...])
    o_ref[...] = jnp.dot(z, w3_ref[...], preferred_element_type=F32) + b3_ref[...]


def _make_deg_kernel(n_pad, n_edges):
    mesh = plsc.VectorSubcoreMesh(core_axis_name="c", subcore_axis_name="s")
    stripe = n_pad // 16
    ept = n_edges // 32
    ke = 40
    nch = ept // ke

    @functools.partial(
        pl.kernel, mesh=mesh,
        out_type=[jax.ShapeDtypeStruct((n_pad,), F32),
                  jax.ShapeDtypeStruct((n_pad,), F32)],
        scratch_types=[
            pltpu.VMEM((ke,), jnp.int32),
            pltpu.VMEM((48,), F32),
            pltpu.VMEM((stripe,), F32),
            pltpu.VMEM_SHARED((n_pad,), F32),
        ],
    )
    def deg_kernel(ed_hbm, d0_hbm, d1_hbm, idx_v, ones_v, zb_v, acc_sh):
        c = lax.axis_index("c")
        s = lax.axis_index("s")
        for i in range(3):
            ones_v[pl.ds(i * 16, 16)] = jnp.ones((16,), F32)
        for i in range(stripe // 16):
            zb_v[pl.ds(i * 16, 16)] = jnp.zeros((16,), F32)
        pltpu.sync_copy(zb_v, acc_sh.at[pl.ds(s * stripe, stripe)])
        plsc.subcore_barrier()

        base = c * (n_edges // 2) + s * ept

        def body(k, carry):
            e0 = base + k * ke
            pltpu.sync_copy(ed_hbm.at[pl.ds(e0, ke)], idx_v)
            pltpu.sync_copy(ones_v.at[pl.ds(0, ke)], acc_sh.at[idx_v],
                            add=True)
            return carry

        lax.fori_loop(0, nch, body, 0)
        plsc.subcore_barrier()

        @pl.when(c == 0)
        def _():
            pltpu.sync_copy(acc_sh.at[pl.ds(s * stripe, stripe)],
                            d0_hbm.at[pl.ds(s * stripe, stripe)])

        @pl.when(c == 1)
        def _():
            pltpu.sync_copy(acc_sh.at[pl.ds(s * stripe, stripe)],
                            d1_hbm.at[pl.ds(s * stripe, stripe)])

    return deg_kernel


def _make_edge_kernel(n_pad, n_edges, hh):
    mesh = plsc.VectorSubcoreMesh(core_axis_name="c", subcore_axis_name="s")
    stripe = n_pad // 16
    ept = n_edges // 16
    ke = 80
    nch = ept // ke

    @functools.partial(
        pl.kernel, mesh=mesh,
        out_type=[jax.ShapeDtypeStruct((n_pad, hh), F32),
                  jax.ShapeDtypeStruct((n_pad, hh), F32)],
        scratch_types=[
            pltpu.VMEM((ke,), jnp.int32),
            pltpu.VMEM((ke,), jnp.int32),
            pltpu.VMEM((ke, hh), F32),
            pltpu.VMEM_SHARED((n_pad, hh), F32),
            pltpu.SemaphoreType.DMA,
        ],
    )
    def edge_kernel(xw0_hbm, xw1_hbm, es_hbm, ed_hbm, s0_hbm, s1_hbm,
                    esc_v, edc_v, rows_v, acc_sh, sem):
        c = lax.axis_index("c")
        s = lax.axis_index("s")

        def zb_body(r, carry):
            for j in range(hh // 16):
                rows_v[r, pl.ds(j * 16, 16)] = jnp.zeros((16,), F32)
            return carry

        lax.fori_loop(0, ke, zb_body, 0)
        for t in range(stripe // ke):
            pltpu.sync_copy(rows_v,
                            acc_sh.at[pl.ds(s * stripe + t * ke, ke)])
        plsc.subcore_barrier()

        def edge_loop(xw_hbm):
            base = s * ept

            def body(k, carry):
                e0 = base + k * ke
                pltpu.sync_copy(es_hbm.at[pl.ds(e0, ke)], esc_v)
                pltpu.sync_copy(ed_hbm.at[pl.ds(e0, ke)], edc_v)
                pltpu.async_copy(xw_hbm.at[esc_v], rows_v, sem).wait()
                pltpu.sync_copy(rows_v, acc_sh.at[edc_v], add=True)
                return carry

            lax.fori_loop(0, nch, body, 0)

        @pl.when(c == 0)
        def _():
            edge_loop(xw0_hbm)

        @pl.when(c == 1)
        def _():
            edge_loop(xw1_hbm)

        plsc.subcore_barrier()

        @pl.when(c == 0)
        def _():
            pltpu.sync_copy(acc_sh.at[pl.ds(s * stripe, stripe)],
                            s0_hbm.at[pl.ds(s * stripe, stripe)])

        @pl.when(c == 1)
        def _():
            pltpu.sync_copy(acc_sh.at[pl.ds(s * stripe, stripe)],
                            s1_hbm.at[pl.ds(s * stripe, stripe)])

    return edge_kernel


def kernel(x, edge_index, batch, W0, b0, Wc, bc, gamma, beta,
           W1, b1, W2, b2, W3, b3):
    n, d = x.shape
    e_cnt = edge_index.shape[1]
    h_dim = W0.shape[1]
    n_layers = Wc.shape[0]
    hh = h_dim // 2
    n_pad = ((n + 255) // 256) * 256
    rb = 1000
    n_row_blocks = n // rb

    es = edge_index[0]
    ed = edge_index[1]
    b0r = b0.reshape(1, h_dim)

    full = lambda shp: pl.BlockSpec(shp, lambda i: tuple(0 for _ in shp))
    rowblk = lambda cols: pl.BlockSpec((rb, cols), lambda i: (i, 0))

    h0 = pl.pallas_call(
        _in_proj_body,
        grid=(n_row_blocks,),
        in_specs=[rowblk(d), full((d, h_dim)), full((1, h_dim))],
        out_specs=rowblk(h_dim),
        out_shape=jax.ShapeDtypeStruct((n, h_dim), F32),
    )(x, W0, b0r)

    deg_k = _make_deg_kernel(n_pad, e_cnt)
    d0p, d1p = deg_k(ed)
    d0 = d0p[:n].reshape(n, 1)
    d1 = d1p[:n].reshape(n, 1)

    xw0, xw1 = pl.pallas_call(
        _mm_split_body,
        grid=(n_row_blocks,),
        in_specs=[rowblk(h_dim), full((h_dim, h_dim)), rowblk(1), rowblk(1)],
        out_specs=[rowblk(hh), rowblk(hh)],
        out_shape=[jax.ShapeDtypeStruct((n, hh), F32),
                   jax.ShapeDtypeStruct((n, hh), F32)],
    )(h0, Wc[0], d0, d1)

    edge_k = _make_edge_kernel(n_pad, e_cnt, hh)

    step_specs = [rowblk(hh), rowblk(hh), rowblk(hh), rowblk(hh),
                  rowblk(1), rowblk(1), full((1, h_dim)), full((1, h_dim)),
                  full((1, h_dim))]

    for l in range(n_layers):
        s0p, s1p = edge_k(xw0, xw1, es, ed)
        s0 = s0p[:n]
        s1 = s1p[:n]
        args = (s0, s1, xw0, xw1, d0, d1, bc[l].reshape(1, h_dim),
                gamma[l].reshape(1, h_dim), beta[l].reshape(1, h_dim))
        if l < n_layers - 1:
            xw0, xw1 = pl.pallas_call(
                _step_body,
                grid=(n_row_blocks,),
                in_specs=step_specs + [full((h_dim, h_dim))],
                out_specs=[rowblk(hh), rowblk(hh)],
                out_shape=[jax.ShapeDtypeStruct((n, hh), F32),
                           jax.ShapeDtypeStruct((n, hh), F32)],
            )(*args, Wc[l + 1])
        else:
            h_fin = pl.pallas_call(
                _post_body,
                grid=(n_row_blocks,),
                in_specs=step_specs,
                out_specs=rowblk(h_dim),
                out_shape=jax.ShapeDtypeStruct((n, h_dim), F32),
            )(*args)

    bpad = 10240
    batch_pad = jnp.pad(batch.astype(jnp.int32), (0, bpad - n),
                        constant_values=NUM_GRAPHS).reshape(bpad // 128, 128)
    sums, maxs, cnts = pl.pallas_call(
        _pool_body,
        grid=(NUM_GRAPHS // 8,),
        in_specs=[full((n, h_dim)), full((bpad // 128, 128))],
        out_specs=[pl.BlockSpec((8, h_dim), lambda i: (i, 0)),
                   pl.BlockSpec((8, h_dim), lambda i: (i, 0)),
                   pl.BlockSpec((8, 1), lambda i: (i, 0))],
        out_shape=[jax.ShapeDtypeStruct((NUM_GRAPHS, h_dim), F32),
                   jax.ShapeDtypeStruct((NUM_GRAPHS, h_dim), F32),
                   jax.ShapeDtypeStruct((NUM_GRAPHS, 1), F32)],
    )(h_fin, batch_pad)

    c_out = W3.shape[1]
    c_pad = 128
    W3p = jnp.pad(W3, ((0, 0), (0, c_pad - c_out)))
    b3p = jnp.pad(b3, (0, c_pad - c_out)).reshape(1, c_pad)
    out = pl.pallas_call(
        _mlp_body,
        grid=(1,),
        in_specs=[full((NUM_GRAPHS, h_dim)), full((NUM_GRAPHS, h_dim)),
                  full((NUM_GRAPHS, 1)), full((2 * h_dim, h_dim)),
                  full((1, h_dim)), full((h_dim, hh)), full((1, hh)),
                  full((hh, c_pad)), full((1, c_pad))],
        out_specs=full((NUM_GRAPHS, c_pad)),
        out_shape=jax.ShapeDtypeStruct((NUM_GRAPHS, c_pad), F32),
    )(sums, maxs, cnts, W1, b1.reshape(1, h_dim), W2, b2.reshape(1, hh),
      W3p, b3p)
    return out[:, :c_out]
